# baseline (speedup 1.0000x reference)
import functools

import jax
import jax.numpy as jnp
from jax import lax
from jax.experimental import pallas as pl
from jax.experimental.pallas import tpu as pltpu
from jax.experimental.pallas import tpu_sc as plsc


def _vq_argmin_body(z_ref, emb_ref, idx_ref, gidx_ref):
    g = pl.program_id(0)
    z = z_ref[0]
    emb = emb_ref[0]
    K = emb.shape[0]
    HW = z.shape[1]

    embs = emb * (-2.0)
    e2 = jnp.sum(emb * emb, axis=1, keepdims=True)
    z2 = jnp.sum(z * z, axis=0, keepdims=True)
    cross2 = jax.lax.dot_general(
        embs, z, (((1,), (0,)), ((), ())),
        preferred_element_type=jnp.float32)
    dist = (e2 + z2) + cross2

    m = jnp.min(dist, axis=0, keepdims=True)
    kiota = jax.lax.broadcasted_iota(jnp.int32, (K, HW), 0)
    idx = jnp.min(jnp.where(dist == m, kiota, K), axis=0,
                  keepdims=True)

    idx_ref[0, 0, :] = idx[0]
    gidx_ref[0, 0, :] = idx[0] + g * K


def _sc_gather(table, gidx, B, D, b_per_w, nc):
    mesh = plsc.VectorSubcoreMesh(core_axis_name="c", subcore_axis_name="s")

    @functools.partial(
        pl.kernel, mesh=mesh,
        out_type=jax.ShapeDtypeStruct((B, D), jnp.float32),
        compiler_params=pltpu.CompilerParams(use_tc_tiling_on_sc=False),
        scratch_types=[
            pltpu.VMEM((b_per_w,), jnp.int32),
            pltpu.VMEM((b_per_w, D), jnp.float32),
            pltpu.SemaphoreType.DMA,
        ],
    )
    def gather_rows(table_hbm, gidx_hbm, out_hbm, idx_v, rows_v, sem):
        wid = lax.axis_index("s") * nc + lax.axis_index("c")
        base = wid * b_per_w
        pltpu.sync_copy(gidx_hbm.at[pl.ds(base, b_per_w)], idx_v)
        pltpu.async_copy(table_hbm.at[idx_v], rows_v, sem).wait()
        pltpu.sync_copy(rows_v, out_hbm.at[pl.ds(base, b_per_w)])

    return gather_rows(table, gidx)


def kernel(feather, codebooks):
    N, C, H, W = feather.shape
    G, K, gs = codebooks.shape
    HW = H * W
    fr = feather.reshape(N, C, HW)

    idx_r, gidx_r = pl.pallas_call(
        _vq_argmin_body,
        grid=(G, N),
        in_specs=[
            pl.BlockSpec((1, gs, HW), lambda g, n: (n, g, 0)),
            pl.BlockSpec((1, K, gs), lambda g, n: (g, 0, 0)),
        ],
        out_specs=[
            pl.BlockSpec((1, 1, HW), lambda g, n: (g * N + n, 0, 0)),
            pl.BlockSpec((1, 1, HW), lambda g, n: (g * N + n, 0, 0)),
        ],
        out_shape=[
            jax.ShapeDtypeStruct((G * N, 1, HW), jnp.int32),
            jax.ShapeDtypeStruct((G * N, 1, HW), jnp.int32),
        ],
    )(fr, codebooks)

    info = plsc.get_sparse_core_info()
    nw = info.num_cores * info.num_subcores
    B = G * N * HW
    rows = _sc_gather(codebooks.reshape(G * K, gs), gidx_r.reshape(B),
                      B, gs, B // nw, info.num_cores)

    zq = (rows.reshape(G, N, HW, gs)
              .transpose(1, 0, 3, 2)
              .reshape(N, C, H, W))
    code_index = idx_r.reshape(G, N, H, W).transpose(1, 0, 2, 3)
    return (zq, zq, code_index)


# speedup vs baseline: 1.4942x; 1.4942x over previous
import jax
import jax.numpy as jnp
from jax.experimental import pallas as pl
from jax.experimental.pallas import tpu as pltpu

_PAIR = 2


def _vq_step(z_ref, emb_ref, zq_ref, idx_ref):
    emb = emb_ref[0]
    K = emb.shape[0]
    HW = z_ref.shape[2]

    embs = emb * (-2.0)
    e2 = jnp.sum(emb * emb, axis=1, keepdims=True)
    kiota = jax.lax.broadcasted_iota(jnp.int32, (K, HW), 0)

    for i in range(_PAIR):
        z = z_ref[i]
        z2 = jnp.sum(z * z, axis=0, keepdims=True)
        cross2 = jax.lax.dot_general(
            embs, z, (((1,), (0,)), ((), ())),
            preferred_element_type=jnp.float32)
        dist = (e2 + z2) + cross2

        m = jnp.min(dist, axis=0, keepdims=True)
        idx = jnp.min(jnp.where(dist == m, kiota, K), axis=0,
                      keepdims=True)
        onehot = (kiota == idx).astype(jnp.float32)
        zq = jax.lax.dot_general(
            emb, onehot, (((0,), (0,)), ((), ())),
            preferred_element_type=jnp.float32)

        zq_ref[i] = zq
        idx_ref[0, i, :] = idx[0]


def kernel(feather, codebooks):
    N, C, H, W = feather.shape
    G, K, gs = codebooks.shape
    HW = H * W
    NP = N // _PAIR
    fr = feather.reshape(N, C, HW)

    def outer(fr_hbm, cb_hbm, zq_hbm, idx_hbm):
        pipeline = pltpu.emit_pipeline(
            _vq_step,
            grid=(G, NP),
            in_specs=[
                pl.BlockSpec((_PAIR, gs, HW), lambda g, n: (n, g, 0)),
                pl.BlockSpec((1, K, gs), lambda g, n: (g, 0, 0)),
            ],
            out_specs=[
                pl.BlockSpec((_PAIR, gs, HW), lambda g, n: (n, g, 0)),
                pl.BlockSpec((1, _PAIR, HW), lambda g, n: (g * NP + n, 0, 0)),
            ],
        )
        pipeline(fr_hbm, cb_hbm, zq_hbm, idx_hbm)

    zq_r, idx_r = pl.pallas_call(
        outer,
        in_specs=[
            pl.BlockSpec(memory_space=pl.ANY),
            pl.BlockSpec(memory_space=pl.ANY),
        ],
        out_specs=[
            pl.BlockSpec(memory_space=pl.ANY),
            pl.BlockSpec(memory_space=pl.ANY),
        ],
        out_shape=[
            jax.ShapeDtypeStruct((N, C, HW), jnp.float32),
            jax.ShapeDtypeStruct((G * NP, _PAIR, HW), jnp.int32),
        ],
    )(fr, codebooks)

    zq = zq_r.reshape(N, C, H, W)
    code_index = idx_r.reshape(G, N, H, W).transpose(1, 0, 2, 3)
    return (zq, zq, code_index)
